# Initial kernel scaffold; baseline (speedup 1.0000x reference)
import jax
import jax.numpy as jnp
from jax.experimental import pallas as pl


def kernel(x, Wr, Wg, Wu, Wd):
    raise NotImplementedError("write your pallas kernel here")


# speedup vs baseline: 5.8027x; 5.8027x over previous
import functools

import jax
import jax.numpy as jnp
from jax import lax
from jax.experimental import pallas as pl
from jax.experimental.pallas import tpu as pltpu
from jax.experimental.pallas import tpu_sc as plsc

H = 768
E = 64
EI = 96
N = 4096
B = 128
NBLK = N // B
T_MAX = NBLK + E - 1
P = T_MAX * B

RB = 512
NW = 32
RPW = N // NW


def _router_body(x_ref, wr_ref, ids_ref):
    logits = lax.dot_general(x_ref[...], wr_ref[...],
                             (((1,), (1,)), ((), ())),
                             preferred_element_type=jnp.float32)
    ids_ref[...] = jnp.argmax(logits, axis=1).astype(jnp.int32)


def _router(x2d, Wr):
    return pl.pallas_call(
        _router_body,
        grid=(N // RB,),
        in_specs=[
            pl.BlockSpec((RB, H), lambda i: (i, 0)),
            pl.BlockSpec((E, H), lambda i: (0, 0)),
        ],
        out_specs=pl.BlockSpec((RB,), lambda i: (i,)),
        out_shape=jax.ShapeDtypeStruct((N,), jnp.int32),
        compiler_params=pltpu.CompilerParams(
            dimension_semantics=("arbitrary",)),
    )(x2d, Wr)


def _dispatch_meta(ids):
    counts = jnp.zeros((E,), jnp.int32).at[ids].add(1)
    blocks_e = (counts + B - 1) // B
    cumblk = jnp.cumsum(blocks_e)
    cumblk_excl = cumblk - blocks_e
    num_items = cumblk[E - 1]

    t_arr = jnp.arange(T_MAX, dtype=jnp.int32)
    ie_raw = jnp.searchsorted(cumblk, t_arr, side="right").astype(jnp.int32)
    ie_last = ie_raw[jnp.maximum(num_items - 1, 0)]
    item_expert = jnp.where(t_arr < num_items, ie_raw, ie_last)
    item_block = jnp.minimum(t_arr, num_items - 1)

    order = jnp.argsort(ids)
    sorted_ids = ids[order]
    start_cum = jnp.cumsum(counts) - counts
    pad_start = cumblk_excl * B
    j = jnp.arange(N, dtype=jnp.int32)
    dst_sorted = pad_start[sorted_ids] + (j - start_cum[sorted_ids])
    dst_token = jnp.zeros((N,), jnp.int32).at[order].set(dst_sorted)
    return item_expert, item_block, num_items.reshape(1), dst_token


def _sc_wid():
    return lax.axis_index("s") * 2 + lax.axis_index("c")


@functools.cache
def _sc_permute_kernels():
    mesh = plsc.VectorSubcoreMesh(core_axis_name="c", subcore_axis_name="s")
    scratch = [
        pltpu.VMEM((RPW,), jnp.int32),
        pltpu.VMEM((RPW, H), jnp.float32),
        pltpu.SemaphoreType.DMA,
    ]

    @functools.partial(
        pl.kernel,
        out_type=jax.ShapeDtypeStruct((P, H), jnp.float32),
        mesh=mesh,
        scratch_types=scratch,
    )
    def sc_scatter(x_hbm, idx_hbm, out_hbm, idx_v, rows_v, sem):
        base = _sc_wid() * RPW
        pltpu.sync_copy(idx_hbm.at[pl.ds(base, RPW)], idx_v)
        pltpu.sync_copy(x_hbm.at[pl.ds(base, RPW)], rows_v)
        pltpu.async_copy(rows_v, out_hbm.at[idx_v], sem).wait()

    @functools.partial(
        pl.kernel,
        out_type=jax.ShapeDtypeStruct((N, H), jnp.float32),
        mesh=mesh,
        scratch_types=scratch,
    )
    def sc_gather(src_hbm, idx_hbm, out_hbm, idx_v, rows_v, sem):
        base = _sc_wid() * RPW
        pltpu.sync_copy(idx_hbm.at[pl.ds(base, RPW)], idx_v)
        pltpu.async_copy(src_hbm.at[idx_v], rows_v, sem).wait()
        pltpu.sync_copy(rows_v, out_hbm.at[pl.ds(base, RPW)])

    return sc_scatter, sc_gather


def _mlp_body(ie_ref, ib_ref, ni_ref, x_ref, wg_ref, wu_ref, wd_ref, o_ref):
    t = pl.program_id(0)

    @pl.when(t < ni_ref[0])
    def _():
        xb = x_ref[...]
        g = lax.dot_general(xb, wg_ref[0], (((1,), (1,)), ((), ())),
                            preferred_element_type=jnp.float32)
        u = lax.dot_general(xb, wu_ref[0], (((1,), (1,)), ((), ())),
                            preferred_element_type=jnp.float32)
        h = g * jax.nn.sigmoid(g) * u
        o_ref[...] = lax.dot_general(h, wd_ref[0], (((1,), (1,)), ((), ())),
                                     preferred_element_type=jnp.float32)


def _grouped_mlp(x_pad, Wg, Wu, Wd, item_expert, item_block, num_items):
    grid_spec = pltpu.PrefetchScalarGridSpec(
        num_scalar_prefetch=3,
        grid=(T_MAX,),
        in_specs=[
            pl.BlockSpec((B, H), lambda t, ie, ib, ni: (ib[t], 0)),
            pl.BlockSpec((1, EI, H), lambda t, ie, ib, ni: (ie[t], 0, 0)),
            pl.BlockSpec((1, EI, H), lambda t, ie, ib, ni: (ie[t], 0, 0)),
            pl.BlockSpec((1, H, EI), lambda t, ie, ib, ni: (ie[t], 0, 0)),
        ],
        out_specs=pl.BlockSpec((B, H), lambda t, ie, ib, ni: (ib[t], 0)),
    )
    return pl.pallas_call(
        _mlp_body,
        grid_spec=grid_spec,
        out_shape=jax.ShapeDtypeStruct((P, H), jnp.float32),
        compiler_params=pltpu.CompilerParams(
            dimension_semantics=("arbitrary",)),
    )(item_expert, item_block, num_items, x_pad, Wg, Wu, Wd)


def kernel(x, Wr, Wg, Wu, Wd):
    x2d = x.reshape(N, H)
    sc_scatter, sc_gather = _sc_permute_kernels()
    ids = _router(x2d, Wr)
    item_expert, item_block, num_items, dst_token = _dispatch_meta(ids)
    x_pad = sc_scatter(x2d, dst_token)
    out_pad = _grouped_mlp(x_pad, Wg, Wu, Wd, item_expert, item_block,
                           num_items)
    out2d = sc_gather(out_pad, dst_token)
    return out2d.reshape(x.shape)


# speedup vs baseline: 11.5168x; 1.9847x over previous
import functools

import jax
import jax.numpy as jnp
from jax import lax
from jax.experimental import pallas as pl
from jax.experimental.pallas import tpu as pltpu
from jax.experimental.pallas import tpu_sc as plsc

H = 768
E = 64
EI = 96
N = 4096
B = 128
NBLK = N // B
T_MAX = NBLK + E - 1
P = T_MAX * B

RB = 512
NW = 32
RPW = N // NW


def _router_body(x_ref, wr_ref, ids_ref):
    logits = lax.dot_general(x_ref[...], wr_ref[...],
                             (((1,), (1,)), ((), ())),
                             preferred_element_type=jnp.float32)
    ids_ref[...] = jnp.argmax(logits, axis=1).astype(jnp.int32)


def _router(x2d, Wr):
    return pl.pallas_call(
        _router_body,
        grid=(N // RB,),
        in_specs=[
            pl.BlockSpec((RB, H), lambda i: (i, 0)),
            pl.BlockSpec((E, H), lambda i: (0, 0)),
        ],
        out_specs=pl.BlockSpec((RB,), lambda i: (i,)),
        out_shape=jax.ShapeDtypeStruct((N,), jnp.int32),
        compiler_params=pltpu.CompilerParams(
            dimension_semantics=("arbitrary",)),
    )(x2d, Wr)


def _meta_body(ids_ref, dst_ref, ie_ref, ib_ref, ni_ref):
    iota_e = lax.broadcasted_iota(jnp.int32, (E, 1), 0)
    tlane = lax.broadcasted_iota(jnp.int32, (1, B), 1)
    tril = (lax.broadcasted_iota(jnp.int32, (B, 1), 0)
            < lax.broadcasted_iota(jnp.int32, (1, B), 1)).astype(jnp.float32)
    tri_e = (lax.broadcasted_iota(jnp.int32, (E, 1), 0)
             >= lax.broadcasted_iota(jnp.int32, (1, E), 1)).astype(jnp.float32)

    def onehot(i):
        idb = ids_ref[pl.ds(i, 1), :]
        return (idb == iota_e).astype(jnp.int32)

    def body_a(i, counts):
        return counts + jnp.sum(onehot(i), axis=1, keepdims=True)

    counts = lax.fori_loop(0, NBLK, body_a, jnp.zeros((E, 1), jnp.int32))

    blocks_e = (counts + B - 1) // B
    cumblk = lax.dot_general(tri_e, blocks_e.astype(jnp.float32),
                             (((1,), (0,)), ((), ())),
                             preferred_element_type=jnp.float32
                             ).astype(jnp.int32)
    cumblk_excl = cumblk - blocks_e
    pad_start = cumblk_excl * B
    ni = jnp.sum(blocks_e)

    ie_raw = jnp.sum((cumblk <= tlane).astype(jnp.int32), axis=0)
    tl = tlane[0]
    ie_at_last = jnp.sum(jnp.where(tl == ni - 1, ie_raw, 0))
    ie_ref[...] = jnp.where(tl < ni, ie_raw, ie_at_last)
    ib_ref[...] = jnp.minimum(tl, ni - 1)
    ni_ref[...] = jnp.full((B,), ni, jnp.int32)

    def body_b(i, counts2):
        oh = onehot(i)
        ohf = oh.astype(jnp.float32)
        rank = lax.dot_general(ohf, tril, (((1,), (0,)), ((), ())),
                               preferred_element_type=jnp.float32
                               ).astype(jnp.int32)
        base = pad_start + counts2
        dst = jnp.sum(oh * (base + rank), axis=0)
        dst_ref[pl.ds(i, 1), :] = dst.reshape(1, B)
        return counts2 + jnp.sum(oh, axis=1, keepdims=True)

    lax.fori_loop(0, NBLK, body_b, jnp.zeros((E, 1), jnp.int32))


def _dispatch_meta(ids):
    dst, ie, ib, ni = pl.pallas_call(
        _meta_body,
        out_shape=[
            jax.ShapeDtypeStruct((NBLK, B), jnp.int32),
            jax.ShapeDtypeStruct((B,), jnp.int32),
            jax.ShapeDtypeStruct((B,), jnp.int32),
            jax.ShapeDtypeStruct((B,), jnp.int32),
        ],
    )(ids.reshape(NBLK, B))
    return ie, ib, ni, dst.reshape(N)


def _sc_wid():
    return lax.axis_index("s") * 2 + lax.axis_index("c")


@functools.cache
def _sc_permute_kernels():
    mesh = plsc.VectorSubcoreMesh(core_axis_name="c", subcore_axis_name="s")
    scratch = [
        pltpu.VMEM((RPW,), jnp.int32),
        pltpu.VMEM((RPW, H), jnp.float32),
        pltpu.SemaphoreType.DMA,
    ]

    @functools.partial(
        pl.kernel,
        out_type=jax.ShapeDtypeStruct((P, H), jnp.float32),
        mesh=mesh,
        scratch_types=scratch,
    )
    def sc_scatter(x_hbm, idx_hbm, out_hbm, idx_v, rows_v, sem):
        base = _sc_wid() * RPW
        pltpu.sync_copy(idx_hbm.at[pl.ds(base, RPW)], idx_v)
        pltpu.sync_copy(x_hbm.at[pl.ds(base, RPW)], rows_v)
        pltpu.async_copy(rows_v, out_hbm.at[idx_v], sem).wait()

    @functools.partial(
        pl.kernel,
        out_type=jax.ShapeDtypeStruct((N, H), jnp.float32),
        mesh=mesh,
        scratch_types=scratch,
    )
    def sc_gather(src_hbm, idx_hbm, out_hbm, idx_v, rows_v, sem):
        base = _sc_wid() * RPW
        pltpu.sync_copy(idx_hbm.at[pl.ds(base, RPW)], idx_v)
        pltpu.async_copy(src_hbm.at[idx_v], rows_v, sem).wait()
        pltpu.sync_copy(rows_v, out_hbm.at[pl.ds(base, RPW)])

    return sc_scatter, sc_gather


def _mlp_body(ie_ref, ib_ref, ni_ref, x_ref, wg_ref, wu_ref, wd_ref, o_ref):
    t = pl.program_id(0)

    @pl.when(t < ni_ref[0])
    def _():
        xb = x_ref[...]
        g = lax.dot_general(xb, wg_ref[...], (((1,), (1,)), ((), ())),
                            preferred_element_type=jnp.float32)
        u = lax.dot_general(xb, wu_ref[...], (((1,), (1,)), ((), ())),
                            preferred_element_type=jnp.float32)
        h = g * jax.nn.sigmoid(g) * u
        o_ref[...] = lax.dot_general(h, wd_ref[...], (((1,), (1,)), ((), ())),
                                     preferred_element_type=jnp.float32)


def _grouped_mlp(x_pad, Wg, Wu, Wd, item_expert, item_block, num_items):
    grid_spec = pltpu.PrefetchScalarGridSpec(
        num_scalar_prefetch=3,
        grid=(T_MAX,),
        in_specs=[
            pl.BlockSpec((B, H), lambda t, ie, ib, ni: (ib[t], 0)),
            pl.BlockSpec((EI, H), lambda t, ie, ib, ni: (ie[t], 0)),
            pl.BlockSpec((EI, H), lambda t, ie, ib, ni: (ie[t], 0)),
            pl.BlockSpec((H, EI), lambda t, ie, ib, ni: (ie[t], 0)),
        ],
        out_specs=pl.BlockSpec((B, H), lambda t, ie, ib, ni: (ib[t], 0)),
    )
    return pl.pallas_call(
        _mlp_body,
        grid_spec=grid_spec,
        out_shape=jax.ShapeDtypeStruct((P, H), jnp.float32),
        compiler_params=pltpu.CompilerParams(
            dimension_semantics=("arbitrary",)),
    )(item_expert, item_block, num_items, x_pad,
      Wg.reshape(E * EI, H), Wu.reshape(E * EI, H), Wd.reshape(E * H, EI))


def kernel(x, Wr, Wg, Wu, Wd):
    x2d = x.reshape(N, H)
    sc_scatter, sc_gather = _sc_permute_kernels()
    ids = _router(x2d, Wr)
    item_expert, item_block, num_items, dst_token = _dispatch_meta(ids)
    x_pad = sc_scatter(x2d, dst_token)
    out_pad = _grouped_mlp(x_pad, Wg, Wu, Wd, item_expert, item_block,
                           num_items)
    out2d = sc_gather(out_pad, dst_token)
    return out2d.reshape(x.shape)
